# baseline (speedup 1.0000x reference)
import functools

import jax
import jax.numpy as jnp
from jax import lax
from jax.experimental import layout
from jax.experimental import pallas as pl
from jax.experimental.pallas import tpu as pltpu
from jax.experimental.pallas import tpu_sc as plsc

D_MODEL = 32
CHUNK = 128
G_PER_STEP = 10
NUM_CORES = 2
NUM_SUBCORES = 16
NUM_WORKERS = NUM_CORES * NUM_SUBCORES
STEP_ROWS = CHUNK * G_PER_STEP


@functools.cache
def _build(n_chunks_per_worker: int, b_per_worker: int, total_rows: int):
    mesh = plsc.VectorSubcoreMesh(
        core_axis_name="c",
        subcore_axis_name="s",
        num_cores=NUM_CORES,
        num_subcores=NUM_SUBCORES,
    )
    n_steps = n_chunks_per_worker // G_PER_STEP
    assert n_steps % 2 == 0

    scratch = [pltpu.VMEM((n_chunks_per_worker, CHUNK), jnp.int32)]
    scratch += [pltpu.VMEM((STEP_ROWS, D_MODEL), jnp.float32) for _ in range(2)]
    scratch += [pltpu.SemaphoreType.DMA for _ in range(4)]

    @functools.partial(
        pl.kernel,
        mesh=mesh,
        out_type=jax.ShapeDtypeStruct((total_rows, D_MODEL), jnp.float32),
        scratch_types=scratch,
        compiler_params=pltpu.CompilerParams(use_tc_tiling_on_sc=False),
    )
    def emb_kernel(idx_hbm, table_hbm, out_hbm, idx_v, buf0, buf1,
                   gsem0, gsem1, ssem0, ssem1):
        bufs = (buf0, buf1)
        gsems = (gsem0, gsem1)
        ssems = (ssem0, ssem1)

        wid = lax.axis_index("s") * NUM_CORES + lax.axis_index("c")
        base = wid * b_per_worker

        pltpu.sync_copy(idx_hbm.at[wid], idx_v)

        def gather(bb, s, k):
            return pltpu.make_async_copy(
                table_hbm.at[idx_v.at[s * G_PER_STEP + k]],
                bufs[bb].at[pl.ds(k * CHUNK, CHUNK)],
                gsems[bb])

        def fire(bb, s):
            for k in range(G_PER_STEP):
                gather(bb, s, k).start()

        def drain(bb, s):
            for k in range(G_PER_STEP):
                gather(bb, s, k).wait()

        def store(bb, s):
            return pltpu.make_async_copy(
                bufs[bb], out_hbm.at[pl.ds(base + s * STEP_ROWS, STEP_ROWS)],
                ssems[bb])

        fire(0, 0)

        def outer(g, carry):
            s0 = 2 * g
            s1 = s0 + 1

            @pl.when(g > 0)
            def _():
                store(1, s1 - 2).wait()
            fire(1, s1)
            drain(0, s0)
            store(0, s0).start()

            @pl.when(s1 + 1 < n_steps)
            def _():
                store(0, s0).wait()
                fire(0, s1 + 1)
            drain(1, s1)
            store(1, s1).start()
            return carry

        lax.fori_loop(0, n_steps // 2, outer, 0)

        store(0, n_steps - 2).wait()
        store(1, n_steps - 1).wait()

    return emb_kernel


def _linear_fmt(ndim, sharding):
    return layout.Format(
        layout.Layout(major_to_minor=tuple(range(ndim)), tiling=()),
        sharding,
    )


@functools.cache
def _jitted():
    sharding = jax.sharding.SingleDeviceSharding(jax.devices()[0])

    def impl(x, table):
        batch, hist = x.shape
        total = batch * hist
        b_per_worker = total // NUM_WORKERS
        n_chunks = b_per_worker // CHUNK
        idx = x.reshape(-1).astype(jnp.int32).reshape(
            NUM_WORKERS, n_chunks, CHUNK)
        idx = layout.with_layout_constraint(
            idx, layout.Layout(major_to_minor=(0, 1, 2), tiling=()))
        out = _build(n_chunks, b_per_worker, total)(idx, table)
        out = layout.with_layout_constraint(
            out, layout.Layout(major_to_minor=(0, 1), tiling=()))
        return out.reshape(batch, hist, D_MODEL)

    return jax.jit(impl, out_shardings=_linear_fmt(3, sharding))


def kernel(x, table):
    return _jitted()(x, table)


# speedup vs baseline: 2.0629x; 2.0629x over previous
import functools

import jax
import jax.numpy as jnp
from jax import lax
from jax.experimental import pallas as pl
from jax.experimental.pallas import tpu as pltpu
from jax.experimental.pallas import tpu_sc as plsc

D_MODEL = 32
NB = 32
NUM_CORES = 2
NUM_SUBCORES = 16
NUM_WORKERS = NUM_CORES * NUM_SUBCORES


@functools.cache
def _build(batch: int, hist: int):
    batches_per_worker = batch // NUM_WORKERS
    n_steps = batches_per_worker // NB
    assert n_steps % 2 == 0 and batches_per_worker % NB == 0

    mesh = plsc.VectorSubcoreMesh(
        core_axis_name="c",
        subcore_axis_name="s",
        num_cores=NUM_CORES,
        num_subcores=NUM_SUBCORES,
    )

    scratch = [pltpu.VMEM((batches_per_worker, hist), jnp.int32)]
    scratch += [pltpu.VMEM((NB, hist, D_MODEL), jnp.float32)
                for _ in range(2)]
    scratch += [pltpu.SemaphoreType.DMA for _ in range(4)]

    @functools.partial(
        pl.kernel,
        mesh=mesh,
        out_type=jax.ShapeDtypeStruct((batch, hist, D_MODEL), jnp.float32),
        scratch_types=scratch,
        compiler_params=pltpu.CompilerParams(use_tc_tiling_on_sc=False),
    )
    def emb_kernel(x_hbm, table_hbm, out_hbm, idx_v, buf0, buf1,
                   gsem0, gsem1, ssem0, ssem1):
        bufs = (buf0, buf1)
        gsems = (gsem0, gsem1)
        ssems = (ssem0, ssem1)

        wid = lax.axis_index("s") * NUM_CORES + lax.axis_index("c")
        bbase = wid * batches_per_worker

        pltpu.sync_copy(x_hbm.at[pl.ds(bbase, batches_per_worker)], idx_v)

        def gather(bb, i, c):
            return pltpu.make_async_copy(
                table_hbm.at[idx_v.at[c]], bufs[bb].at[i], gsems[bb])

        def fire(bb, s):
            def body(i, carry):
                gather(bb, i, s * NB + i).start()
                return carry
            lax.fori_loop(0, NB, body, 0)

        def drain(bb, s):
            def body(i, carry):
                gather(bb, i, s * NB + i).wait()
                return carry
            lax.fori_loop(0, NB, body, 0)

        def store(bb, s):
            return pltpu.make_async_copy(
                bufs[bb], out_hbm.at[pl.ds(bbase + s * NB, NB)], ssems[bb])

        fire(0, 0)

        def outer(g, carry):
            s0 = 2 * g
            s1 = s0 + 1

            @pl.when(g > 0)
            def _():
                store(1, s1 - 2).wait()
            fire(1, s1)
            drain(0, s0)
            store(0, s0).start()

            @pl.when(s1 + 1 < n_steps)
            def _():
                store(0, s0).wait()
                fire(0, s1 + 1)
            drain(1, s1)
            store(1, s1).start()
            return carry

        lax.fori_loop(0, n_steps // 2, outer, 0)

        store(0, n_steps - 2).wait()
        store(1, n_steps - 1).wait()

    return emb_kernel


@jax.jit
def kernel(x, table):
    batch, hist = x.shape
    return _build(batch, hist)(x.astype(jnp.int32), table)
